# baseline (speedup 1.0000x reference)
import functools

import jax
import jax.numpy as jnp
from jax import lax
from jax.experimental import pallas as pl
from jax.experimental.pallas import tpu as pltpu
from jax.experimental.pallas import tpu_sc as plsc

H = 128
LANES = 16
NCOL = H // LANES
T = 128
EPS = 1e-12

_GATHER_DNUMS = lax.GatherDimensionNumbers(
    offset_dims=(), collapsed_slice_dims=(0,), start_index_map=(0,))


def _lane_perm(v, idx):
    return lax.gather(
        v, idx[:, None], _GATHER_DNUMS, (1,),
        indices_are_sorted=False, unique_indices=False,
        mode=lax.GatherScatterMode.PROMISE_IN_BOUNDS)


def _bcast_sum(v):
    idx = lax.iota(jnp.int32, LANES)
    for k in (1, 2, 4, 8):
        v = v + _lane_perm(v, idx ^ k)
    return v


def _rsqrt(x):
    i = lax.bitcast_convert_type(x, jnp.int32)
    y = lax.bitcast_convert_type(jnp.int32(0x5F3759DF) - (i >> 1), jnp.float32)
    for _ in range(2):
        y = y * (1.5 - 0.5 * x * y * y)
    return y


def kernel(input_ids, table, gamma, beta):
    B, S = input_ids.shape
    N = B * S

    info = plsc.get_sparse_core_info()
    NC, NS = info.num_cores, info.num_subcores
    NW = NC * NS
    per_w = N // NW
    nchunk = per_w // T
    ids3 = input_ids.reshape(NW, nchunk, T)

    mesh = plsc.VectorSubcoreMesh(core_axis_name="c", subcore_axis_name="s")

    @functools.partial(
        pl.kernel,
        mesh=mesh,
        out_type=jax.ShapeDtypeStruct((N, H), jnp.float32),
        scratch_types=[
            pltpu.VMEM((nchunk, T), jnp.int32),
            pltpu.VMEM((3, T, H), jnp.float32),
            pltpu.VMEM((H,), jnp.float32),
            pltpu.VMEM((H,), jnp.float32),
            pltpu.SemaphoreType.DMA((3,)),
            pltpu.SemaphoreType.DMA((3,)),
        ],
    )
    def _k(ids_hbm, table_hbm, gamma_hbm, beta_hbm, out_hbm,
           idx_v, rows_v, gamma_v, beta_v, gsem, osem):
        wid = lax.axis_index("s") * NC + lax.axis_index("c")
        base0 = wid * per_w
        pltpu.sync_copy(gamma_hbm, gamma_v)
        pltpu.sync_copy(beta_hbm, beta_v)
        pltpu.sync_copy(ids_hbm.at[wid], idx_v)

        pltpu.async_copy(table_hbm.at[idx_v.at[0]], rows_v.at[0], gsem.at[0])

        gvecs = tuple(gamma_v[pl.ds(LANES * c, LANES)] for c in range(NCOL))
        bvecs = tuple(beta_v[pl.ds(LANES * c, LANES)] for c in range(NCOL))

        def one_token(rows_b, t, gv, bv):
            xs = [rows_b[t, pl.ds(LANES * c, LANES)] for c in range(NCOL)]
            s = xs[0]
            ss = xs[0] * xs[0]
            for c in range(1, NCOL):
                s = s + xs[c]
                ss = ss + xs[c] * xs[c]
            m = _bcast_sum(s) * (1.0 / H)
            var = _bcast_sum(ss) * (1.0 / H) - m * m
            r = _rsqrt(jnp.maximum(var, 0.0) + EPS)
            for c in range(NCOL):
                rows_b[t, pl.ds(LANES * c, LANES)] = (xs[c] - m) * (r * gv[c]) + bv[c]

        def out_copy(b, ci):
            return pltpu.make_async_copy(
                rows_v.at[b], out_hbm.at[pl.ds(base0 + ci * T, T)], osem.at[b])

        def chunk_body(ci, carry):
            gv, bv = carry
            b = lax.rem(ci, 3)
            nb = lax.rem(ci + 1, 3)
            rows_b = rows_v.at[b]

            @pl.when(ci >= 2)
            def _drain():
                out_copy(nb, ci - 2).wait()

            @pl.when(ci + 1 < nchunk)
            def _prefetch():
                pltpu.async_copy(
                    table_hbm.at[idx_v.at[ci + 1]], rows_v.at[nb], gsem.at[nb])

            pltpu.make_async_copy(
                table_hbm.at[idx_v.at[ci]], rows_b, gsem.at[b]).wait()

            @plsc.parallel_loop(0, T, unroll=8)
            def token_body(t):
                one_token(rows_b, t, gv, bv)
            pltpu.async_copy(
                rows_b, out_hbm.at[pl.ds(base0 + ci * T, T)], osem.at[b])
            return carry

        lax.fori_loop(0, nchunk, chunk_body, (gvecs, bvecs))
        out_copy((nchunk - 2) % 3, nchunk - 2).wait()
        out_copy((nchunk - 1) % 3, nchunk - 1).wait()

    out = _k(ids3, table, gamma, beta)
    return out.reshape(B, S, H)


# speedup vs baseline: 2.8036x; 2.8036x over previous
import functools

import jax
import jax.numpy as jnp
from jax import lax
from jax.experimental import pallas as pl
from jax.experimental.pallas import tpu as pltpu
from jax.experimental.pallas import tpu_sc as plsc

H = 128
LANES = 16
NCOL = H // LANES
T = 128
EPS = 1e-12

_GATHER_DNUMS = lax.GatherDimensionNumbers(
    offset_dims=(), collapsed_slice_dims=(0,), start_index_map=(0,))


def _lane_perm(v, idx):
    return lax.gather(
        v, idx[:, None], _GATHER_DNUMS, (1,),
        indices_are_sorted=False, unique_indices=False,
        mode=lax.GatherScatterMode.PROMISE_IN_BOUNDS)


def _bcast_sum(v):
    idx = lax.iota(jnp.int32, LANES)
    for k in (1, 2, 4, 8):
        v = v + _lane_perm(v, idx ^ k)
    return v


def _rsqrt(x):
    i = lax.bitcast_convert_type(x, jnp.int32)
    y = lax.bitcast_convert_type(jnp.int32(0x5F3759DF) - (i >> 1), jnp.float32)
    for _ in range(2):
        y = y * (1.5 - 0.5 * x * y * y)
    return y


def kernel(input_ids, table, gamma, beta):
    B, S = input_ids.shape
    N = B * S

    info = plsc.get_sparse_core_info()
    NC, NS = info.num_cores, info.num_subcores
    NW = NC * NS
    per_w = N // NW
    nchunk = per_w // T
    ids3 = input_ids.reshape(NW, nchunk, T)

    mesh = plsc.VectorSubcoreMesh(core_axis_name="c", subcore_axis_name="s")

    @functools.partial(
        pl.kernel,
        mesh=mesh,
        out_type=jax.ShapeDtypeStruct((N, H), jnp.float32),
        scratch_types=[
            pltpu.VMEM((nchunk, T), jnp.int32),
            pltpu.VMEM((3, T, H), jnp.float32),
            pltpu.VMEM((H,), jnp.float32),
            pltpu.VMEM((H,), jnp.float32),
            pltpu.SemaphoreType.DMA((3,)),
            pltpu.SemaphoreType.DMA((3,)),
        ],
    )
    def _k(ids_hbm, table_hbm, gamma_hbm, beta_hbm, out_hbm,
           idx_v, rows_v, gamma_v, beta_v, gsem, osem):
        wid = lax.axis_index("s") * NC + lax.axis_index("c")
        base0 = wid * per_w
        pltpu.sync_copy(gamma_hbm, gamma_v)
        pltpu.sync_copy(beta_hbm, beta_v)
        pltpu.sync_copy(ids_hbm.at[wid], idx_v)

        pltpu.async_copy(table_hbm.at[idx_v.at[0]], rows_v.at[0], gsem.at[0])

        gvecs = tuple(gamma_v[pl.ds(LANES * c, LANES)] for c in range(NCOL))
        bvecs = tuple(beta_v[pl.ds(LANES * c, LANES)] for c in range(NCOL))

        def one_token(rows_b, t, gv, bv):
            xs = [rows_b[t, pl.ds(LANES * c, LANES)] for c in range(NCOL)]
            s = xs[0]
            ss = xs[0] * xs[0]
            for c in range(1, NCOL):
                s = s + xs[c]
                ss = ss + xs[c] * xs[c]
            m = _bcast_sum(s) * (1.0 / H)
            var = _bcast_sum(ss) * (1.0 / H) - m * m
            r = _rsqrt(jnp.maximum(var, 0.0) + EPS)
            for c in range(NCOL):
                rows_b[t, pl.ds(LANES * c, LANES)] = (xs[c] - m) * (r * gv[c]) + bv[c]

        def out_copy(b, ci):
            return pltpu.make_async_copy(
                rows_v.at[b], out_hbm.at[pl.ds(base0 + ci * T, T)], osem.at[b])

        def chunk_body(ci, carry):
            gv, bv = carry
            b = lax.rem(ci, 3)
            nb = lax.rem(ci + 1, 3)
            rows_b = rows_v.at[b]

            @pl.when(ci >= 2)
            def _drain():
                out_copy(nb, ci - 2).wait()

            @pl.when(ci + 1 < nchunk)
            def _prefetch():
                pltpu.async_copy(
                    table_hbm.at[idx_v.at[ci + 1]], rows_v.at[nb], gsem.at[nb])

            pltpu.make_async_copy(
                table_hbm.at[idx_v.at[ci]], rows_b, gsem.at[b]).wait()

            @plsc.parallel_loop(0, T, unroll=2)
            def token_body(t):
                one_token(rows_b, t, gv, bv)
            pltpu.async_copy(
                rows_b, out_hbm.at[pl.ds(base0 + ci * T, T)], osem.at[b])
            return carry

        lax.fori_loop(0, nchunk, chunk_body, (gvecs, bvecs))
        out_copy((nchunk - 2) % 3, nchunk - 2).wait()
        out_copy((nchunk - 1) % 3, nchunk - 1).wait()

    out = _k(ids3, table, gamma, beta)
    return out.reshape(B, S, H)
